# baseline (device time: 88688 ns/iter reference)
import jax
import jax.numpy as jnp
from jax import lax
from jax.experimental import pallas as pl
from jax.experimental.pallas import tpu as pltpu

N_DEV = 8
BLOCK_M = 512


def kernel(x):
    m_per, n = x.shape
    n_blocks = m_per // BLOCK_M

    def body(x_ref, out_ref, carry_ref, prefix_ref, acc_ref, send_sem, recv_sem):
        b = pl.program_id(0)
        my = lax.axis_index("i")

        @pl.when(b == 0)
        def _():
            carry_ref[...] = jnp.ones((1, n), jnp.float32)

        g = BLOCK_M // 8
        z = x_ref[...].reshape(g, 8, n)
        for s in (1, 2, 4):
            pad = jnp.ones((g, s, n), jnp.float32)
            z = z * jnp.concatenate([pad, z[:, :-s, :]], axis=1)
        t = z[:, 7, :]
        s = 1
        while s < g:
            pad = jnp.ones((s, n), jnp.float32)
            t = t * jnp.concatenate([pad, t[:-s, :]], axis=0)
            s *= 2
        factor = (
            jnp.concatenate([jnp.ones((1, n), jnp.float32), t[:-1, :]], axis=0)
            * carry_ref[...]
        )
        z = z * factor[:, None, :]
        out_ref[pl.ds(b * BLOCK_M, BLOCK_M), :] = z.reshape(BLOCK_M, n)
        carry_ref[...] = t[g - 1 : g, :] * carry_ref[...]

        @pl.when(b == n_blocks - 1)
        def _():
            left = my - 1
            right = my + 1

            @pl.when(my == 0)
            def _():
                prefix_ref[...] = jnp.ones((1, n), jnp.float32)

            @pl.when(my > 0)
            def _():
                recv = pltpu.make_async_remote_copy(
                    src_ref=acc_ref,
                    dst_ref=prefix_ref,
                    send_sem=send_sem,
                    recv_sem=recv_sem,
                    device_id=(left,),
                    device_id_type=pl.DeviceIdType.MESH,
                )
                recv.wait_recv()

            acc_ref[...] = prefix_ref[...] * carry_ref[...]

            @pl.when(my < N_DEV - 1)
            def _():
                send = pltpu.make_async_remote_copy(
                    src_ref=acc_ref,
                    dst_ref=prefix_ref,
                    send_sem=send_sem,
                    recv_sem=recv_sem,
                    device_id=(right,),
                    device_id_type=pl.DeviceIdType.MESH,
                )
                send.start()
                send.wait_send()

            out_ref[...] = out_ref[...] * prefix_ref[...]

    return pl.pallas_call(
        body,
        grid=(n_blocks,),
        in_specs=[pl.BlockSpec((BLOCK_M, n), lambda b: (b, 0))],
        out_specs=pl.BlockSpec((m_per, n), lambda b: (0, 0)),
        out_shape=jax.ShapeDtypeStruct((m_per, n), jnp.float32),
        scratch_shapes=[
            pltpu.VMEM((1, n), jnp.float32),
            pltpu.VMEM((1, n), jnp.float32),
            pltpu.VMEM((1, n), jnp.float32),
            pltpu.SemaphoreType.DMA,
            pltpu.SemaphoreType.DMA,
        ],
        compiler_params=pltpu.CompilerParams(
            dimension_semantics=("arbitrary",),
            vmem_limit_bytes=60 * 1024 * 1024,
        ),
    )(x)
